# baseline (speedup 1.0000x reference)
import functools

import jax
import jax.numpy as jnp
from jax import lax
from jax.experimental import pallas as pl
from jax.experimental.pallas import tpu as pltpu
from jax.experimental.pallas import tpu_sc as plsc

_B, _L, _V, _D, _C, _R = 1024, 20, 1000, 128, 128, 7
_S = _B * 2
_NC, _NS = 2, 16
_NW = _NC * _NS
_SEG_W = _S // _NW
_BAT_W = _SEG_W // 2
_LANES = 16

_SEG_RD = 16
_BAT_RD = _SEG_RD // 2
_ROWS_RD = _SEG_RD * _L
_GS = (128, 128, 64)
_NRD = _SEG_W // _SEG_RD


def _sc_segment_sums(idx_flat, voc_bf):
    mesh = plsc.VectorSubcoreMesh(core_axis_name="c", subcore_axis_name="s")

    @functools.partial(
        pl.kernel,
        mesh=mesh,
        out_type=(
            jax.ShapeDtypeStruct((_B, _D), jnp.bfloat16),
            jax.ShapeDtypeStruct((_B, _D), jnp.bfloat16),
        ),
        scratch_types=[
            pltpu.VMEM((_SEG_W * _L,), jnp.int32),
            pltpu.VMEM((_NRD, _ROWS_RD, _D), jnp.bfloat16),
            pltpu.VMEM((_BAT_W, _D), jnp.bfloat16),
            pltpu.VMEM((_BAT_W, _D), jnp.bfloat16),
            pltpu.SemaphoreType.DMA,
            pltpu.SemaphoreType.DMA,
            pltpu.SemaphoreType.DMA,
            pltpu.SemaphoreType.DMA,
        ],
        compiler_params=pltpu.CompilerParams(use_tc_tiling_on_sc=False),
    )
    def body(idx_hbm, voc_hbm, oute_hbm, outo_hbm, idx_v, rows_v, acce_v, acco_v,
             sem0, sem1, sem2, sem3):
        wid = lax.axis_index("s") * _NC + lax.axis_index("c")
        pltpu.sync_copy(idx_hbm.at[pl.ds(wid * _SEG_W * _L, _SEG_W * _L)], idx_v)
        sems = (sem0, sem1, sem2, sem3)

        def fire(g, b):
            ro = 0
            for gl in _GS:
                pltpu.async_copy(
                    voc_hbm.at[idx_v.at[pl.ds(g * _ROWS_RD + ro, gl)]],
                    rows_v.at[b, pl.ds(ro, gl)],
                    sems[b],
                )
                ro += gl

        def drain(b):
            ro = 0
            for gl in _GS:
                pltpu.make_async_copy(
                    voc_hbm.at[pl.ds(0, gl)],
                    rows_v.at[b, pl.ds(ro, gl)],
                    sems[b],
                ).wait()
                ro += gl

        def compute(g, b):
            def batch(bt, carry2):
                for half, acc_ref in ((0, acce_v), (1, acco_v)):
                    base = (bt * 2 + half) * _L
                    for j in range(_D // 32):
                        sl = pl.ds(j * 32, 32)
                        t = [
                            rows_v[b, base + 2 * r, sl]
                            + rows_v[b, base + 2 * r + 1, sl]
                            for r in range(_L // 2)
                        ]
                        while len(t) > 1:
                            nxt = [
                                t[2 * i] + t[2 * i + 1] for i in range(len(t) // 2)
                            ]
                            if len(t) % 2:
                                nxt.append(t[-1])
                            t = nxt
                        acc_ref[g * _BAT_RD + bt, sl] = t[0]
                return carry2

            lax.fori_loop(0, _BAT_RD, batch, 0)

        for g in range(_NRD):
            fire(g, g)
        for g in range(_NRD):
            drain(g)
            compute(g, g)
        pltpu.sync_copy(acce_v, oute_hbm.at[pl.ds(wid * _BAT_W, _BAT_W)])
        pltpu.sync_copy(acco_v, outo_hbm.at[pl.ds(wid * _BAT_W, _BAT_W)])

    return body(idx_flat, voc_bf)


def _mlp_body(xe_ref, xo_ref, w1_ref, b1_ref, w2_ref, b2_ref, o_ref):
    w1 = w1_ref[...]
    nt = (((1,), (1,)), ((), ()))
    h = (
        lax.dot_general(xe_ref[...], w1[:, :_D], nt, preferred_element_type=jnp.float32)
        + lax.dot_general(xo_ref[...], w1[:, _D:], nt, preferred_element_type=jnp.float32)
        + b1_ref[...]
    )
    h = jnp.where(h >= 0, h, 0.01 * h)
    logits = (
        lax.dot_general(h, w2_ref[...], nt, preferred_element_type=jnp.float32)
        + b2_ref[...]
    )
    mx = jnp.max(logits, axis=1, keepdims=True)
    lse = jnp.log(jnp.sum(jnp.exp(logits - mx), axis=1, keepdims=True)) + mx
    o_ref[...] = logits - lse


def kernel(inputs, voc, cpr_w, cpr_b, sm_w, sm_b):
    idx_flat = inputs.astype(jnp.int32).reshape(_S * _L)
    voc_bf = voc.astype(jnp.bfloat16)
    sums_e, sums_o = _sc_segment_sums(idx_flat, voc_bf)

    out = pl.pallas_call(
        _mlp_body,
        out_shape=jax.ShapeDtypeStruct((_B, _R), jnp.float32),
    )(sums_e, sums_o, cpr_w, cpr_b.reshape(1, _C), sm_w, sm_b.reshape(1, _R))
    return out


# speedup vs baseline: 1.0481x; 1.0481x over previous
import functools

import jax
import jax.numpy as jnp
from jax import lax
from jax.experimental import pallas as pl
from jax.experimental.pallas import tpu as pltpu
from jax.experimental.pallas import tpu_sc as plsc

_B, _L, _V, _D, _C, _R = 1024, 20, 1000, 128, 128, 7
_S = _B * 2
_NC, _NS = 2, 16
_NW = _NC * _NS
_SEG_W = _S // _NW
_BAT_W = _SEG_W // 2
_LANES = 16

_SEG_RD = 16
_BAT_RD = _SEG_RD // 2
_ROWS_RD = _SEG_RD * _L
_GS = (128, 128, 64)
_NRD = _SEG_W // _SEG_RD


def _sc_segment_sums(idx_flat, voc_bf):
    mesh = plsc.VectorSubcoreMesh(core_axis_name="c", subcore_axis_name="s")

    @functools.partial(
        pl.kernel,
        mesh=mesh,
        out_type=(
            jax.ShapeDtypeStruct((_B, _D), jnp.bfloat16),
            jax.ShapeDtypeStruct((_B, _D), jnp.bfloat16),
        ),
        scratch_types=[
            pltpu.VMEM((_SEG_W * _L,), jnp.int32),
            pltpu.VMEM((2, _ROWS_RD, _D), jnp.bfloat16),
            pltpu.VMEM((_BAT_W, _D), jnp.bfloat16),
            pltpu.VMEM((_BAT_W, _D), jnp.bfloat16),
            pltpu.SemaphoreType.DMA,
            pltpu.SemaphoreType.DMA,
        ],
        compiler_params=pltpu.CompilerParams(use_tc_tiling_on_sc=False),
    )
    def body(idx_hbm, voc_hbm, oute_hbm, outo_hbm, idx_v, rows_v, acce_v, acco_v,
             sem0, sem1):
        wid = lax.axis_index("s") * _NC + lax.axis_index("c")
        pltpu.sync_copy(idx_hbm.at[pl.ds(wid * _SEG_W * _L, _SEG_W * _L)], idx_v)
        sems = (sem0, sem1)

        def fire(g, b):
            ro = 0
            for gl in _GS:
                pltpu.async_copy(
                    voc_hbm.at[idx_v.at[pl.ds(g * _ROWS_RD + ro, gl)]],
                    rows_v.at[b, pl.ds(ro, gl)],
                    sems[b],
                )
                ro += gl

        def drain(b):
            ro = 0
            for gl in _GS:
                pltpu.make_async_copy(
                    voc_hbm.at[pl.ds(0, gl)],
                    rows_v.at[b, pl.ds(ro, gl)],
                    sems[b],
                ).wait()
                ro += gl

        def compute(g, b):
            def batch(bt, carry2):
                for half, acc_ref in ((0, acce_v), (1, acco_v)):
                    base = (bt * 2 + half) * _L
                    for j in range(_D // 32):
                        sl = pl.ds(j * 32, 32)
                        t = [
                            rows_v[b, base + 2 * r, sl]
                            + rows_v[b, base + 2 * r + 1, sl]
                            for r in range(_L // 2)
                        ]
                        while len(t) > 1:
                            nxt = [
                                t[2 * i] + t[2 * i + 1] for i in range(len(t) // 2)
                            ]
                            if len(t) % 2:
                                nxt.append(t[-1])
                            t = nxt
                        acc_ref[g * _BAT_RD + bt, sl] = t[0]
                return carry2

            lax.fori_loop(0, _BAT_RD, batch, 0)

        fire(0, 0)
        fire(1, 1)
        for g in range(_NRD):
            b = g % 2
            drain(b)
            compute(g, b)
            if g + 2 < _NRD:
                fire(g + 2, b)
        pltpu.sync_copy(acce_v, oute_hbm.at[pl.ds(wid * _BAT_W, _BAT_W)])
        pltpu.sync_copy(acco_v, outo_hbm.at[pl.ds(wid * _BAT_W, _BAT_W)])

    return body(idx_flat, voc_bf)


def _mlp_body(xe_ref, xo_ref, w1_ref, b1_ref, w2_ref, b2_ref, o_ref):
    w1 = w1_ref[...]
    nt = (((1,), (1,)), ((), ()))
    h = (
        lax.dot_general(xe_ref[...], w1[:, :_D], nt, preferred_element_type=jnp.float32)
        + lax.dot_general(xo_ref[...], w1[:, _D:], nt, preferred_element_type=jnp.float32)
        + b1_ref[...]
    )
    h = jnp.where(h >= 0, h, 0.01 * h)
    logits = (
        lax.dot_general(h, w2_ref[...], nt, preferred_element_type=jnp.float32)
        + b2_ref[...]
    )
    mx = jnp.max(logits, axis=1, keepdims=True)
    lse = jnp.log(jnp.sum(jnp.exp(logits - mx), axis=1, keepdims=True)) + mx
    o_ref[...] = logits - lse


def kernel(inputs, voc, cpr_w, cpr_b, sm_w, sm_b):
    idx_flat = inputs.astype(jnp.int32).reshape(_S * _L)
    voc_bf = voc.astype(jnp.bfloat16)
    sums_e, sums_o = _sc_segment_sums(idx_flat, voc_bf)

    out = pl.pallas_call(
        _mlp_body,
        out_shape=jax.ShapeDtypeStruct((_B, _R), jnp.float32),
    )(sums_e, sums_o, cpr_w, cpr_b.reshape(1, _C), sm_w, sm_b.reshape(1, _R))
    return out
